# Initial kernel scaffold; baseline (speedup 1.0000x reference)
import jax
import jax.numpy as jnp
from jax.experimental import pallas as pl


def kernel(z, emb_w):
    raise NotImplementedError("write your pallas kernel here")


# speedup vs baseline: 1.1935x; 1.1935x over previous
import functools

import jax
import jax.numpy as jnp
from jax import lax
from jax.experimental import pallas as pl
from jax.experimental.pallas import tpu as pltpu
from jax.experimental.pallas import tpu_sc as plsc

N_CODES = 8192
E_DIM = 256
BETA = 0.25
N_TOKENS = 8192
TILE_T = 256


def _vq_tc_body(z_ref, embt_ref, enc_ref, idx_ref, loss_ref, perp_ref,
                counts_ref, dsum_ref):
    step = pl.program_id(0)
    nsteps = pl.num_programs(0)

    @pl.when(step == 0)
    def _init():
        counts_ref[...] = jnp.zeros_like(counts_ref)
        dsum_ref[...] = jnp.zeros_like(dsum_ref)

    z = z_ref[...]
    embt = embt_ref[...]
    a = jnp.sum(z * z, axis=1, keepdims=True)
    b = jnp.sum(embt * embt, axis=0, keepdims=True)
    s = lax.dot_general(z, embt, (((1,), (0,)), ((), ())),
                        preferred_element_type=jnp.float32)
    d = (a + b) - 2.0 * s
    dmin2 = jnp.min(d, axis=1, keepdims=True)
    iota = lax.broadcasted_iota(jnp.int32, (TILE_T, N_CODES), 1)
    cand = jnp.where(d == dmin2, iota, jnp.int32(N_CODES))
    idx = jnp.min(cand, axis=1).astype(jnp.int32)
    dmin = dmin2[:, 0]
    onehot = (iota == idx[:, None]).astype(jnp.float32)
    enc_ref[...] = onehot
    idx_ref[...] = idx[:, None]
    counts_ref[...] += jnp.sum(onehot, axis=0, keepdims=True)
    dsum_ref[...] += jnp.sum(dmin).reshape(1, 1)

    @pl.when(step == nsteps - 1)
    def _finalize():
        total = jnp.float32(N_TOKENS * E_DIM)
        loss_ref[...] = (1.0 + BETA) * (dsum_ref[...] / total)
        p = counts_ref[...] * (1.0 / N_TOKENS)
        perp_ref[...] = jnp.exp(-jnp.sum(p * jnp.log(p + 1e-10))).reshape(1, 1)


def _vq_distances_argmin(z_flat, embt):
    grid = (N_TOKENS // TILE_T,)
    return pl.pallas_call(
        _vq_tc_body,
        grid=grid,
        in_specs=[
            pl.BlockSpec((TILE_T, E_DIM), lambda i: (i, 0)),
            pl.BlockSpec((E_DIM, N_CODES), lambda i: (0, 0)),
        ],
        out_specs=[
            pl.BlockSpec((TILE_T, N_CODES), lambda i: (i, 0)),
            pl.BlockSpec((TILE_T, 1), lambda i: (i, 0)),
            pl.BlockSpec((1, 1), lambda i: (0, 0)),
            pl.BlockSpec((1, 1), lambda i: (0, 0)),
        ],
        out_shape=[
            jax.ShapeDtypeStruct((N_TOKENS, N_CODES), jnp.float32),
            jax.ShapeDtypeStruct((N_TOKENS, 1), jnp.int32),
            jax.ShapeDtypeStruct((1, 1), jnp.float32),
            jax.ShapeDtypeStruct((1, 1), jnp.float32),
        ],
        scratch_shapes=[
            pltpu.VMEM((1, N_CODES), jnp.float32),
            pltpu.VMEM((1, 1), jnp.float32),
        ],
        compiler_params=pltpu.CompilerParams(
            dimension_semantics=("arbitrary",),
        ),
    )(z_flat, embt)


@functools.lru_cache(maxsize=1)
def _make_sc_gather():
    info = plsc.get_sparse_core_info()
    nc, ns = info.num_cores, info.num_subcores
    nw = nc * ns
    b_per_w = N_TOKENS // nw
    mesh = plsc.VectorSubcoreMesh(core_axis_name="c", subcore_axis_name="s")

    @functools.partial(
        pl.kernel, mesh=mesh,
        out_type=jax.ShapeDtypeStruct((N_TOKENS, E_DIM), jnp.float32),
        scratch_types=[
            pltpu.VMEM((b_per_w,), jnp.int32),
            pltpu.VMEM((b_per_w, E_DIM), jnp.float32),
            pltpu.SemaphoreType.DMA,
        ],
    )
    def gather(table_hbm, idx_hbm, out_hbm, idx_v, rows_v, sem):
        wid = lax.axis_index("s") * nc + lax.axis_index("c")
        base = wid * b_per_w
        pltpu.sync_copy(idx_hbm.at[pl.ds(base, b_per_w)], idx_v)
        pltpu.async_copy(table_hbm.at[idx_v], rows_v, sem).wait()
        pltpu.sync_copy(rows_v, out_hbm.at[pl.ds(base, b_per_w)])

    return gather


def kernel(z, emb_w):
    zp = jnp.transpose(z, (0, 2, 3, 1))
    z_flat = zp.reshape(-1, E_DIM)
    embt = emb_w.T
    enc, idx2, loss, perp = _vq_distances_argmin(z_flat, embt)
    z_q = _make_sc_gather()(emb_w, idx2.reshape(-1)).reshape(zp.shape)
    z_q_st = zp + (z_q - zp)
    z_q_out = jnp.transpose(z_q_st, (0, 3, 1, 2))
    return (z_q_out, loss[0, 0], perp[0, 0], enc, idx2)


# speedup vs baseline: 1.2499x; 1.0473x over previous
import functools

import jax
import jax.numpy as jnp
from jax import lax
from jax.experimental import pallas as pl
from jax.experimental.pallas import tpu as pltpu
from jax.experimental.pallas import tpu_sc as plsc

N_CODES = 8192
E_DIM = 256
BETA = 0.25
N_TOKENS = 8192
TILE_T = 256


def _vq_tc_body(z_ref, embt2_ref, enc_ref, idx_ref, loss_ref, perp_ref,
                counts_ref, dsum_ref, b_ref):
    step = pl.program_id(0)
    nsteps = pl.num_programs(0)

    @pl.when(step == 0)
    def _init():
        counts_ref[...] = jnp.zeros_like(counts_ref)
        dsum_ref[...] = jnp.zeros_like(dsum_ref)
        eh = 0.5 * embt2_ref[...]
        b_ref[...] = jnp.sum(eh * eh, axis=0, keepdims=True)

    z = z_ref[...]
    embt2 = embt2_ref[...]
    a = jnp.sum(z * z, axis=1, keepdims=True)
    s2 = lax.dot_general(z, embt2, (((1,), (0,)), ((), ())),
                         preferred_element_type=jnp.float32)
    d = (a + b_ref[...]) - s2
    dmin2 = jnp.min(d, axis=1, keepdims=True)
    iota = lax.broadcasted_iota(jnp.int32, (TILE_T, N_CODES), 1)
    cand = jnp.where(d == dmin2, iota, jnp.int32(N_CODES))
    idx = jnp.min(cand, axis=1).astype(jnp.int32)
    dmin = dmin2[:, 0]
    onehot = (cand == idx[:, None]).astype(jnp.float32)
    enc_ref[...] = onehot
    idx_ref[...] = idx[:, None]
    counts_ref[...] += jnp.sum(onehot, axis=0, keepdims=True)
    dsum_ref[...] += jnp.sum(dmin).reshape(1, 1)

    @pl.when(step == nsteps - 1)
    def _finalize():
        total = jnp.float32(N_TOKENS * E_DIM)
        loss_ref[...] = (1.0 + BETA) * (dsum_ref[...] / total)
        p = counts_ref[...] * (1.0 / N_TOKENS)
        perp_ref[...] = jnp.exp(-jnp.sum(p * jnp.log(p + 1e-10))).reshape(1, 1)


def _vq_distances_argmin(z_flat, embt):
    grid = (N_TOKENS // TILE_T,)
    return pl.pallas_call(
        _vq_tc_body,
        grid=grid,
        in_specs=[
            pl.BlockSpec((TILE_T, E_DIM), lambda i: (i, 0)),
            pl.BlockSpec((E_DIM, N_CODES), lambda i: (0, 0)),
        ],
        out_specs=[
            pl.BlockSpec((TILE_T, N_CODES), lambda i: (i, 0)),
            pl.BlockSpec((TILE_T, 1), lambda i: (i, 0)),
            pl.BlockSpec((1, 1), lambda i: (0, 0)),
            pl.BlockSpec((1, 1), lambda i: (0, 0)),
        ],
        out_shape=[
            jax.ShapeDtypeStruct((N_TOKENS, N_CODES), jnp.float32),
            jax.ShapeDtypeStruct((N_TOKENS, 1), jnp.int32),
            jax.ShapeDtypeStruct((1, 1), jnp.float32),
            jax.ShapeDtypeStruct((1, 1), jnp.float32),
        ],
        scratch_shapes=[
            pltpu.VMEM((1, N_CODES), jnp.float32),
            pltpu.VMEM((1, 1), jnp.float32),
            pltpu.VMEM((1, N_CODES), jnp.float32),
        ],
        compiler_params=pltpu.CompilerParams(
            dimension_semantics=("arbitrary",),
        ),
    )(z_flat, embt)


@functools.lru_cache(maxsize=1)
def _make_sc_gather():
    info = plsc.get_sparse_core_info()
    nc, ns = info.num_cores, info.num_subcores
    nw = nc * ns
    b_per_w = N_TOKENS // nw
    mesh = plsc.VectorSubcoreMesh(core_axis_name="c", subcore_axis_name="s")

    @functools.partial(
        pl.kernel, mesh=mesh,
        out_type=jax.ShapeDtypeStruct((N_TOKENS, E_DIM), jnp.float32),
        scratch_types=[
            pltpu.VMEM((b_per_w,), jnp.int32),
            pltpu.VMEM((b_per_w, E_DIM), jnp.float32),
            pltpu.SemaphoreType.DMA,
        ],
    )
    def gather(table_hbm, idx_hbm, out_hbm, idx_v, rows_v, sem):
        wid = lax.axis_index("s") * nc + lax.axis_index("c")
        base = wid * b_per_w
        pltpu.sync_copy(idx_hbm.at[pl.ds(base, b_per_w)], idx_v)
        pltpu.async_copy(table_hbm.at[idx_v], rows_v, sem).wait()
        pltpu.sync_copy(rows_v, out_hbm.at[pl.ds(base, b_per_w)])

    return gather


def kernel(z, emb_w):
    zp = jnp.transpose(z, (0, 2, 3, 1))
    z_flat = zp.reshape(-1, E_DIM)
    embt2 = (emb_w * 2.0).T
    enc, idx2, loss, perp = _vq_distances_argmin(z_flat, embt2)
    z_q = _make_sc_gather()(emb_w, idx2.reshape(-1)).reshape(zp.shape)
    z_q_st = zp + (z_q - zp)
    z_q_out = jnp.transpose(z_q_st, (0, 3, 1, 2))
    return (z_q_out, loss[0, 0], perp[0, 0], enc, idx2)
